# baseline (speedup 1.0000x reference)
import jax
import jax.numpy as jnp
from jax import lax
from jax.experimental import pallas as pl
from jax.experimental.pallas import tpu as pltpu
from jax.experimental.pallas import tpu_sc as plsc

NUM_WORKERS = 32
NUM_SUBCORES = 16
NUM_CHUNKS = 4
CHUNK = 128
LANES = 16
EMBED = 128
UNROLL = 8
NUM_USERS_ = 100000
NUM_ITEMS_ = 1000000


def _body(uids_hbm, iids_hbm, utab_hbm, itab_hbm, ubias_hbm, ibias_hbm,
          out_hbm, uid_v, iid_v, urows0, urows1, irows0, irows1,
          ubv, ibv, sums,
          sem_u0, sem_u1, sem_i0, sem_i1, sem_b):
    cid_ax = lax.axis_index("c")
    sid = lax.axis_index("s")
    wid = sid * 2 + cid_ax
    pltpu.sync_copy(uids_hbm.at[wid], uid_v)
    pltpu.sync_copy(iids_hbm.at[wid], iid_v)

    riota = lax.iota(jnp.int32, LANES)
    urows = (urows0, urows1)
    irows = (irows0, irows1)
    sem_u = (sem_u0, sem_u1)
    sem_i = (sem_i0, sem_i1)

    def start(c, b):
        return (
            pltpu.async_copy(utab_hbm.at[uid_v.at[c]], urows[b], sem_u[b]),
            pltpu.async_copy(itab_hbm.at[iid_v.at[c]], irows[b], sem_i[b]),
        )

    cps = start(0, 0)
    for c in range(NUM_CHUNKS):
        b = c % 2
        nxt = start(c + 1, 1 - b) if c + 1 < NUM_CHUNKS else None

        cb_u = pltpu.async_copy(ubias_hbm.at[0].at[uid_v.at[c]], ubv, sem_b)
        cb_i = pltpu.async_copy(ibias_hbm.at[0].at[iid_v.at[c]], ibv, sem_b)
        with jax.named_scope(f"wait_rows_{c}"):
            for cp in cps:
                cp.wait()

        def gbody(g, carry, b=b):
            accs = []
            for r in range(LANES):
                row = g * LANES + r
                prods = [urows[b][row, pl.ds(k * LANES, LANES)]
                         * irows[b][row, pl.ds(k * LANES, LANES)]
                         for k in range(EMBED // LANES)]
                while len(prods) > 1:
                    prods = [prods[i] + prods[i + 1]
                             for i in range(0, len(prods) - 1, 2)] + (
                                 [prods[-1]] if len(prods) % 2 else [])
                accs.append(prods[0])
            cur = accs
            for s in range(4):
                st = 1 << s
                mask = (riota & st) == 0
                perm = riota ^ st
                nxt = []
                for j in range(len(cur) // 2):
                    a, bb = cur[2 * j], cur[2 * j + 1]
                    a_sh = a.at[perm].get(mode="promise_in_bounds")
                    b_sh = bb.at[perm].get(mode="promise_in_bounds")
                    nxt.append(jnp.where(mask, a, b_sh) + jnp.where(mask, a_sh, bb))
                cur = nxt
            acc = (cur[0] + ubv[pl.ds(g * LANES, LANES)]
                   + ibv[pl.ds(g * LANES, LANES)])
            sums[pl.ds(g * LANES, LANES)] = acc
            return carry

        with jax.named_scope(f"wait_bias_{c}"):
            cb_u.wait()
            cb_i.wait()
        with jax.named_scope(f"compute_{c}"):
            lax.fori_loop(0, CHUNK // LANES, gbody, 0)

        pltpu.sync_copy(sums, out_hbm.at[wid, c])
        cps = nxt


@jax.jit
def _sc_call(uids, iids, utab, itab, ubias, ibias):
    mesh = plsc.VectorSubcoreMesh(core_axis_name="c", subcore_axis_name="s")
    return pl.kernel(
        _body,
        out_type=jax.ShapeDtypeStruct((NUM_WORKERS, NUM_CHUNKS, CHUNK), jnp.float32),
        mesh=mesh,
        compiler_params=pltpu.CompilerParams(needs_layout_passes=False),
        scratch_types=[
            pltpu.VMEM((NUM_CHUNKS, CHUNK), jnp.int32),
            pltpu.VMEM((NUM_CHUNKS, CHUNK), jnp.int32),
            pltpu.VMEM((CHUNK, EMBED), jnp.float32),
            pltpu.VMEM((CHUNK, EMBED), jnp.float32),
            pltpu.VMEM((CHUNK, EMBED), jnp.float32),
            pltpu.VMEM((CHUNK, EMBED), jnp.float32),
            pltpu.VMEM((CHUNK,), jnp.float32),
            pltpu.VMEM((CHUNK,), jnp.float32),
            pltpu.VMEM((CHUNK,), jnp.float32),
            pltpu.SemaphoreType.DMA,
            pltpu.SemaphoreType.DMA,
            pltpu.SemaphoreType.DMA,
            pltpu.SemaphoreType.DMA,
            pltpu.SemaphoreType.DMA,
        ],
    )(uids, iids, utab, itab, ubias, ibias)


def kernel(user_ids, item_ids, user_table, item_table, user_bias, item_bias):
    batch = user_ids.shape[0]
    uids = user_ids.astype(jnp.int32).reshape(NUM_WORKERS, NUM_CHUNKS, CHUNK)
    iids = item_ids.astype(jnp.int32).reshape(NUM_WORKERS, NUM_CHUNKS, CHUNK)
    out = _sc_call(uids, iids, user_table, item_table,
                   jnp.transpose(user_bias, (1, 0)),
                   jnp.transpose(item_bias, (1, 0)))
    return out.reshape(batch, 1)


# speedup vs baseline: 1.0290x; 1.0290x over previous
import jax
import jax.numpy as jnp
from jax import lax
from jax.experimental import pallas as pl
from jax.experimental.pallas import tpu as pltpu
from jax.experimental.pallas import tpu_sc as plsc

NUM_WORKERS = 32
NUM_SUBCORES = 16
NUM_CHUNKS = 4
CHUNK = 128
LANES = 16
EMBED = 128
UNROLL = 8
NUM_USERS_ = 100000
NUM_ITEMS_ = 1000000


def _body(uids_hbm, iids_hbm, utab_hbm, itab_hbm, ubias_hbm, ibias_hbm,
          out_hbm, uid_v, iid_v, urows0, urows1, irows0, irows1,
          ubv0, ubv1, ibv0, ibv1, sums,
          sem_u0, sem_u1, sem_i0, sem_i1, sem_b0, sem_b1):
    cid_ax = lax.axis_index("c")
    sid = lax.axis_index("s")
    wid = sid * 2 + cid_ax
    pltpu.sync_copy(uids_hbm.at[wid], uid_v)
    pltpu.sync_copy(iids_hbm.at[wid], iid_v)

    riota = lax.iota(jnp.int32, LANES)
    urows = (urows0, urows1)
    irows = (irows0, irows1)
    sem_u = (sem_u0, sem_u1)
    sem_i = (sem_i0, sem_i1)
    sem_b = (sem_b0, sem_b1)
    ubv = (ubv0, ubv1)
    ibv = (ibv0, ibv1)

    def start(c, b):
        return (
            pltpu.async_copy(utab_hbm.at[uid_v.at[c]], urows[b], sem_u[b]),
            pltpu.async_copy(itab_hbm.at[iid_v.at[c]], irows[b], sem_i[b]),
            pltpu.async_copy(ubias_hbm.at[0].at[uid_v.at[c]], ubv[b], sem_b[b]),
            pltpu.async_copy(ibias_hbm.at[0].at[iid_v.at[c]], ibv[b], sem_b[b]),
        )

    cps = start(0, 0)
    for c in range(NUM_CHUNKS):
        b = c % 2
        nxt = start(c + 1, 1 - b) if c + 1 < NUM_CHUNKS else None

        with jax.named_scope(f"wait_rows_{c}"):
            for cp in cps:
                cp.wait()

        def gbody(g, carry, b=b):
            accs = []
            for r in range(LANES):
                row = g * LANES + r
                prods = [urows[b][row, pl.ds(k * LANES, LANES)]
                         * irows[b][row, pl.ds(k * LANES, LANES)]
                         for k in range(EMBED // LANES)]
                while len(prods) > 1:
                    prods = [prods[i] + prods[i + 1]
                             for i in range(0, len(prods) - 1, 2)] + (
                                 [prods[-1]] if len(prods) % 2 else [])
                accs.append(prods[0])
            cur = accs
            for s in range(4):
                st = 1 << s
                mask = (riota & st) == 0
                perm = riota ^ st
                nxt = []
                for j in range(len(cur) // 2):
                    a, bb = cur[2 * j], cur[2 * j + 1]
                    a_sh = a.at[perm].get(mode="promise_in_bounds")
                    b_sh = bb.at[perm].get(mode="promise_in_bounds")
                    nxt.append(jnp.where(mask, a, b_sh) + jnp.where(mask, a_sh, bb))
                cur = nxt
            acc = (cur[0] + ubv[b][pl.ds(g * LANES, LANES)]
                   + ibv[b][pl.ds(g * LANES, LANES)])
            sums[pl.ds(g * LANES, LANES)] = acc
            return carry

        with jax.named_scope(f"compute_{c}"):
            plsc.parallel_loop(0, CHUNK // LANES)(
                lambda g: gbody(g, 0))

        pltpu.sync_copy(sums, out_hbm.at[wid, c])
        cps = nxt


@jax.jit
def _sc_call(uids, iids, utab, itab, ubias, ibias):
    mesh = plsc.VectorSubcoreMesh(core_axis_name="c", subcore_axis_name="s")
    return pl.kernel(
        _body,
        out_type=jax.ShapeDtypeStruct((NUM_WORKERS, NUM_CHUNKS, CHUNK), jnp.float32),
        mesh=mesh,
        compiler_params=pltpu.CompilerParams(needs_layout_passes=False),
        scratch_types=[
            pltpu.VMEM((NUM_CHUNKS, CHUNK), jnp.int32),
            pltpu.VMEM((NUM_CHUNKS, CHUNK), jnp.int32),
            pltpu.VMEM((CHUNK, EMBED), jnp.float32),
            pltpu.VMEM((CHUNK, EMBED), jnp.float32),
            pltpu.VMEM((CHUNK, EMBED), jnp.float32),
            pltpu.VMEM((CHUNK, EMBED), jnp.float32),
            pltpu.VMEM((CHUNK,), jnp.float32),
            pltpu.VMEM((CHUNK,), jnp.float32),
            pltpu.VMEM((CHUNK,), jnp.float32),
            pltpu.VMEM((CHUNK,), jnp.float32),
            pltpu.VMEM((CHUNK,), jnp.float32),
            pltpu.SemaphoreType.DMA,
            pltpu.SemaphoreType.DMA,
            pltpu.SemaphoreType.DMA,
            pltpu.SemaphoreType.DMA,
            pltpu.SemaphoreType.DMA,
            pltpu.SemaphoreType.DMA,
        ],
    )(uids, iids, utab, itab, ubias, ibias)


def kernel(user_ids, item_ids, user_table, item_table, user_bias, item_bias):
    batch = user_ids.shape[0]
    uids = user_ids.astype(jnp.int32).reshape(NUM_WORKERS, NUM_CHUNKS, CHUNK)
    iids = item_ids.astype(jnp.int32).reshape(NUM_WORKERS, NUM_CHUNKS, CHUNK)
    out = _sc_call(uids, iids, user_table, item_table,
                   jnp.transpose(user_bias, (1, 0)),
                   jnp.transpose(item_bias, (1, 0)))
    return out.reshape(batch, 1)


# speedup vs baseline: 1.1030x; 1.0719x over previous
import jax
import jax.numpy as jnp
from jax import lax
from jax.experimental import pallas as pl
from jax.experimental.pallas import tpu as pltpu
from jax.experimental.pallas import tpu_sc as plsc

NUM_WORKERS = 32
NUM_CHUNKS = 4
CHUNK = 128
LANES = 16
EMBED = 128


def _body(uids_hbm, iids_hbm, utab_hbm, itab_hbm, ubias_hbm, ibias_hbm,
          out_hbm, uid_v, iid_v, urows0, urows1, irows0, irows1,
          ubv0, ubv1, ibv0, ibv1, sums,
          sem_u0, sem_u1, sem_i0, sem_i1, sem_b0, sem_b1):
    cid_ax = lax.axis_index("c")
    sid = lax.axis_index("s")
    wid = sid * 2 + cid_ax
    pltpu.sync_copy(uids_hbm.at[wid], uid_v)
    pltpu.sync_copy(iids_hbm.at[wid], iid_v)

    riota = lax.iota(jnp.int32, LANES)
    urows = (urows0, urows1)
    irows = (irows0, irows1)
    sem_u = (sem_u0, sem_u1)
    sem_i = (sem_i0, sem_i1)
    sem_b = (sem_b0, sem_b1)
    ubv = (ubv0, ubv1)
    ibv = (ibv0, ibv1)

    def start(c, b):
        pltpu.async_copy(utab_hbm.at[uid_v.at[c]], urows[b], sem_u[b])
        pltpu.async_copy(itab_hbm.at[iid_v.at[c]], irows[b], sem_i[b])
        pltpu.async_copy(ubias_hbm.at[0].at[uid_v.at[c]], ubv[b], sem_b[b])
        pltpu.async_copy(ibias_hbm.at[0].at[iid_v.at[c]], ibv[b], sem_b[b])

    def wait(c, b):
        pltpu.make_async_copy(utab_hbm.at[uid_v.at[c]], urows[b], sem_u[b]).wait()
        pltpu.make_async_copy(itab_hbm.at[iid_v.at[c]], irows[b], sem_i[b]).wait()
        pltpu.make_async_copy(ubias_hbm.at[0].at[uid_v.at[c]], ubv[b], sem_b[b]).wait()
        pltpu.make_async_copy(ibias_hbm.at[0].at[iid_v.at[c]], ibv[b], sem_b[b]).wait()

    def compute(b):
        def gbody(g):
            accs = []
            for r in range(LANES):
                row = g * LANES + r
                prods = [urows[b][row, pl.ds(k * LANES, LANES)]
                         * irows[b][row, pl.ds(k * LANES, LANES)]
                         for k in range(EMBED // LANES)]
                while len(prods) > 1:
                    prods = [prods[i] + prods[i + 1]
                             for i in range(0, len(prods) - 1, 2)] + (
                                 [prods[-1]] if len(prods) % 2 else [])
                accs.append(prods[0])
            cur = accs
            for s in range(4):
                st = 1 << s
                mask = (riota & st) == 0
                perm = riota ^ st
                nxt = []
                for j in range(len(cur) // 2):
                    a, bb = cur[2 * j], cur[2 * j + 1]
                    a_sh = a.at[perm].get(mode="promise_in_bounds")
                    b_sh = bb.at[perm].get(mode="promise_in_bounds")
                    nxt.append(jnp.where(mask, a, b_sh) + jnp.where(mask, a_sh, bb))
                cur = nxt
            acc = (cur[0] + ubv[b][pl.ds(g * LANES, LANES)]
                   + ibv[b][pl.ds(g * LANES, LANES)])
            sums[pl.ds(g * LANES, LANES)] = acc

        plsc.parallel_loop(0, CHUNK // LANES)(gbody)

    start(0, 0)
    start(1, 1)

    def super_body(c0, carry):
        for b in range(2):
            c = c0 * 2 + b
            with jax.named_scope(f"wait_{b}"):
                wait(c, b)
            with jax.named_scope(f"compute_{b}"):
                compute(b)

            @pl.when(c0 == 0)
            def _():
                start(c + 2, b)

            pltpu.sync_copy(sums, out_hbm.at[wid, c])
        return carry

    lax.fori_loop(0, NUM_CHUNKS // 2, super_body, 0)


@jax.jit
def _sc_call(uids, iids, utab, itab, ubias, ibias):
    mesh = plsc.VectorSubcoreMesh(core_axis_name="c", subcore_axis_name="s")
    return pl.kernel(
        _body,
        out_type=jax.ShapeDtypeStruct((NUM_WORKERS, NUM_CHUNKS, CHUNK), jnp.float32),
        mesh=mesh,
        compiler_params=pltpu.CompilerParams(needs_layout_passes=False),
        scratch_types=[
            pltpu.VMEM((NUM_CHUNKS, CHUNK), jnp.int32),
            pltpu.VMEM((NUM_CHUNKS, CHUNK), jnp.int32),
            pltpu.VMEM((CHUNK, EMBED), jnp.float32),
            pltpu.VMEM((CHUNK, EMBED), jnp.float32),
            pltpu.VMEM((CHUNK, EMBED), jnp.float32),
            pltpu.VMEM((CHUNK, EMBED), jnp.float32),
            pltpu.VMEM((CHUNK,), jnp.float32),
            pltpu.VMEM((CHUNK,), jnp.float32),
            pltpu.VMEM((CHUNK,), jnp.float32),
            pltpu.VMEM((CHUNK,), jnp.float32),
            pltpu.VMEM((CHUNK,), jnp.float32),
            pltpu.SemaphoreType.DMA,
            pltpu.SemaphoreType.DMA,
            pltpu.SemaphoreType.DMA,
            pltpu.SemaphoreType.DMA,
            pltpu.SemaphoreType.DMA,
            pltpu.SemaphoreType.DMA,
        ],
    )(uids, iids, utab, itab, ubias, ibias)


def kernel(user_ids, item_ids, user_table, item_table, user_bias, item_bias):
    batch = user_ids.shape[0]
    uids = user_ids.astype(jnp.int32).reshape(NUM_WORKERS, NUM_CHUNKS, CHUNK)
    iids = item_ids.astype(jnp.int32).reshape(NUM_WORKERS, NUM_CHUNKS, CHUNK)
    out = _sc_call(uids, iids, user_table, item_table,
                   jnp.transpose(user_bias, (1, 0)),
                   jnp.transpose(item_bias, (1, 0)))
    return out.reshape(batch, 1)


# speedup vs baseline: 1.2654x; 1.1473x over previous
import jax
import jax.numpy as jnp
from jax import lax
from jax.experimental import pallas as pl
from jax.experimental.pallas import tpu as pltpu
from jax.experimental.pallas import tpu_sc as plsc

NUM_WORKERS = 32
NUM_CHUNKS = 4
CHUNK = 128
LANES = 16
EMBED = 128


def _body(uids_hbm, iids_hbm, utab_hbm, itab_hbm, ubias_hbm, ibias_hbm,
          out_hbm, uid_v, iid_v, urows0, urows1, irows0, irows1,
          ubv0, ubv1, ibv0, ibv1, sums,
          sem_u0, sem_u1, sem_i0, sem_i1, sem_b0, sem_b1):
    cid_ax = lax.axis_index("c")
    sid = lax.axis_index("s")
    wid = sid * 2 + cid_ax
    pltpu.sync_copy(uids_hbm.at[wid], uid_v)
    pltpu.sync_copy(iids_hbm.at[wid], iid_v)

    riota = lax.iota(jnp.int32, LANES)
    urows = (urows0, urows1)
    irows = (irows0, irows1)
    sem_u = (sem_u0, sem_u1)
    sem_i = (sem_i0, sem_i1)
    sem_b = (sem_b0, sem_b1)
    ubv = (ubv0, ubv1)
    ibv = (ibv0, ibv1)

    def start(c, b):
        pltpu.async_copy(utab_hbm.at[uid_v.at[c]], urows[b], sem_u[b])
        pltpu.async_copy(itab_hbm.at[iid_v.at[c]], irows[b], sem_i[b])
        pltpu.async_copy(ubias_hbm.at[0].at[uid_v.at[c]], ubv[b], sem_b[b])
        pltpu.async_copy(ibias_hbm.at[0].at[iid_v.at[c]], ibv[b], sem_b[b])

    def wait(c, b):
        pltpu.make_async_copy(utab_hbm.at[uid_v.at[c]], urows[b], sem_u[b]).wait()
        pltpu.make_async_copy(itab_hbm.at[iid_v.at[c]], irows[b], sem_i[b]).wait()
        pltpu.make_async_copy(ubias_hbm.at[0].at[uid_v.at[c]], ubv[b], sem_b[b]).wait()
        pltpu.make_async_copy(ibias_hbm.at[0].at[iid_v.at[c]], ibv[b], sem_b[b]).wait()

    def compute(b):
        def gbody(g):
            accs = []
            for r in range(LANES):
                row = g * LANES + r
                acc0 = (urows[b][row, pl.ds(0, LANES)]
                        * irows[b][row, pl.ds(0, LANES)])
                acc1 = (urows[b][row, pl.ds(LANES, LANES)]
                        * irows[b][row, pl.ds(LANES, LANES)])
                for k in range(2, EMBED // LANES, 2):
                    acc0 = acc0 + (urows[b][row, pl.ds(k * LANES, LANES)]
                                   * irows[b][row, pl.ds(k * LANES, LANES)])
                    acc1 = acc1 + (urows[b][row, pl.ds((k + 1) * LANES, LANES)]
                                   * irows[b][row, pl.ds((k + 1) * LANES, LANES)])
                accs.append(acc0 + acc1)
            cur = accs
            for s in range(4):
                st = 1 << s
                mask = (riota & st) == 0
                perm = riota ^ st
                nxt = []
                for j in range(len(cur) // 2):
                    a, bb = cur[2 * j], cur[2 * j + 1]
                    a_sh = a.at[perm].get(mode="promise_in_bounds")
                    b_sh = bb.at[perm].get(mode="promise_in_bounds")
                    nxt.append(jnp.where(mask, a, b_sh) + jnp.where(mask, a_sh, bb))
                cur = nxt
            acc = (cur[0] + ubv[b][pl.ds(g * LANES, LANES)]
                   + ibv[b][pl.ds(g * LANES, LANES)])
            sums[pl.ds(g * LANES, LANES)] = acc

        plsc.parallel_loop(0, CHUNK // LANES, unroll=2)(gbody)

    start(0, 0)
    start(1, 1)

    def super_body(c0, carry):
        for b in range(2):
            c = c0 * 2 + b
            with jax.named_scope(f"wait_{b}"):
                wait(c, b)
            with jax.named_scope(f"compute_{b}"):
                compute(b)

            @pl.when(c0 == 0)
            def _():
                start(c + 2, b)

            pltpu.sync_copy(sums, out_hbm.at[wid, c])
        return carry

    lax.fori_loop(0, NUM_CHUNKS // 2, super_body, 0)


@jax.jit
def _sc_call(uids, iids, utab, itab, ubias, ibias):
    mesh = plsc.VectorSubcoreMesh(core_axis_name="c", subcore_axis_name="s")
    return pl.kernel(
        _body,
        out_type=jax.ShapeDtypeStruct((NUM_WORKERS, NUM_CHUNKS, CHUNK), jnp.float32),
        mesh=mesh,
        compiler_params=pltpu.CompilerParams(needs_layout_passes=False),
        scratch_types=[
            pltpu.VMEM((NUM_CHUNKS, CHUNK), jnp.int32),
            pltpu.VMEM((NUM_CHUNKS, CHUNK), jnp.int32),
            pltpu.VMEM((CHUNK, EMBED), jnp.float32),
            pltpu.VMEM((CHUNK, EMBED), jnp.float32),
            pltpu.VMEM((CHUNK, EMBED), jnp.float32),
            pltpu.VMEM((CHUNK, EMBED), jnp.float32),
            pltpu.VMEM((CHUNK,), jnp.float32),
            pltpu.VMEM((CHUNK,), jnp.float32),
            pltpu.VMEM((CHUNK,), jnp.float32),
            pltpu.VMEM((CHUNK,), jnp.float32),
            pltpu.VMEM((CHUNK,), jnp.float32),
            pltpu.SemaphoreType.DMA,
            pltpu.SemaphoreType.DMA,
            pltpu.SemaphoreType.DMA,
            pltpu.SemaphoreType.DMA,
            pltpu.SemaphoreType.DMA,
            pltpu.SemaphoreType.DMA,
        ],
    )(uids, iids, utab, itab, ubias, ibias)


def kernel(user_ids, item_ids, user_table, item_table, user_bias, item_bias):
    batch = user_ids.shape[0]
    uids = user_ids.astype(jnp.int32).reshape(NUM_WORKERS, NUM_CHUNKS, CHUNK)
    iids = item_ids.astype(jnp.int32).reshape(NUM_WORKERS, NUM_CHUNKS, CHUNK)
    out = _sc_call(uids, iids, user_table, item_table,
                   jnp.transpose(user_bias, (1, 0)),
                   jnp.transpose(item_bias, (1, 0)))
    return out.reshape(batch, 1)
